# baseline (speedup 1.0000x reference)
import functools

import jax
import jax.numpy as jnp
from jax import lax
from jax.experimental import pallas as pl
from jax.experimental.pallas import tpu as pltpu
from jax.experimental.pallas import tpu_sc as plsc

N_NODES = 10000
D = 128
DP = D // 2
NC = 2
NS = 16
NW = NC * NS
CHUNK = 128
N_CHUNKS = 80
N_PHASES = 2
PH_CHUNKS = N_CHUNKS // N_PHASES
E_PAD = NW * N_CHUNKS * CHUNK
N_ACC = 10240
ROWS_PER_TILE = N_ACC // NS
ZROWS = 128


def _sc_scatter(T, src, dst, ew):
    mesh = plsc.VectorSubcoreMesh(
        core_axis_name="c", subcore_axis_name="s",
        num_cores=NC, num_subcores=NS)

    @functools.partial(
        pl.kernel,
        out_type=jax.ShapeDtypeStruct((NC, N_ACC, D), jnp.bfloat16),
        mesh=mesh,
        scratch_types=[
            pltpu.VMEM((PH_CHUNKS, CHUNK), jnp.int32),
            pltpu.VMEM((PH_CHUNKS, CHUNK), jnp.int32),
            pltpu.VMEM((PH_CHUNKS, CHUNK), jnp.uint32),
            pltpu.VMEM((CHUNK, D), jnp.bfloat16),
            pltpu.VMEM((CHUNK, D), jnp.bfloat16),
            pltpu.VMEM((CHUNK, D), jnp.bfloat16),
            pltpu.VMEM((CHUNK, D), jnp.bfloat16),
            pltpu.VMEM_SHARED((N_ACC, D), jnp.bfloat16),
            pltpu.VMEM_SHARED((N_ACC, D), jnp.bfloat16),
            pltpu.SemaphoreType.DMA,
            pltpu.SemaphoreType.DMA,
            pltpu.SemaphoreType.DMA,
            pltpu.SemaphoreType.DMA,
        ],
        compiler_params=pltpu.CompilerParams(use_tc_tiling_on_sc=False,
                                             needs_layout_passes=False),
    )
    def k(t_hbm, src_hbm, dst_hbm, ew_hbm, out_hbm,
          src_v, dst_v, ew_v, g0, g1, s0, s1, tbl, acc,
          sem_g0, sem_g1, sem_s0, sem_s1):
        gbufs = (g0, g1)
        sbufs = (s0, s1)
        sems_g = (sem_g0, sem_g1)
        sems_s = (sem_s0, sem_s1)
        c = lax.axis_index("c")
        s = lax.axis_index("s")
        gwid = c * NS + s
        base = s * ROWS_PER_TILE

        pltpu.sync_copy(t_hbm.at[pl.ds(base, ROWS_PER_TILE)],
                        tbl.at[pl.ds(base, ROWS_PER_TILE)])

        def zrow(i, carry):
            for v in range(D // 32):
                s0[i, pl.ds(32 * v, 32)] = jnp.zeros((32,), jnp.bfloat16)
            return carry
        lax.fori_loop(0, ZROWS, zrow, 0)
        for t in range(ROWS_PER_TILE // ZROWS):
            pltpu.sync_copy(s0, acc.at[pl.ds(base + t * ZROWS, ZROWS)])
        plsc.subcore_barrier()

        def scale(j, src_buf, dst_buf):
            def group(g, gcarry):
                wv = ew_v[j, pl.ds(g * 16, 16)]
                ws = [plsc.bitcast(jnp.full((16,), wv[i], jnp.uint32),
                                   jnp.bfloat16)
                      for i in range(16)]

                def blk(v, bcarry):
                    psl = pl.ds(v * 32, 32)
                    for i in range(16):
                        e = g * 16 + i
                        dst_buf[e, psl] = src_buf[e, psl] * ws[i]
                    return bcarry
                lax.fori_loop(0, D // 32, blk, 0)
                return gcarry
            lax.fori_loop(0, CHUNK // 16, group, 0)

        for phase in range(N_PHASES):
            p0 = phase * PH_CHUNKS
            pltpu.sync_copy(src_hbm.at[gwid, pl.ds(p0, PH_CHUNKS)], src_v)
            pltpu.sync_copy(dst_hbm.at[gwid, pl.ds(p0, PH_CHUNKS)], dst_v)
            pltpu.sync_copy(ew_hbm.at[gwid, pl.ds(p0, PH_CHUNKS)], ew_v)

            srcs = (tbl, t_hbm)
            for b in range(2):
                pltpu.async_copy(srcs[b].at[src_v.at[b]], gbufs[b],
                                 sems_g[b])

            def pair(q, carry):
                for b in range(2):
                    j = 2 * q + b
                    jn = jnp.minimum(j + 2, PH_CHUNKS - 1)

                    pltpu.make_async_copy(
                        srcs[b].at[src_v.at[j]], gbufs[b], sems_g[b]).wait()

                    @pl.when(j >= 2)
                    def _():
                        pltpu.make_async_copy(
                            sbufs[b], acc.at[dst_v.at[j]], sems_s[b]).wait()
                    scale(j, gbufs[b], sbufs[b])
                    pltpu.async_copy(sbufs[b], acc.at[dst_v.at[j]],
                                     sems_s[b], add=True)
                    pltpu.async_copy(srcs[b].at[src_v.at[jn]], gbufs[b],
                                     sems_g[b])
                return carry
            lax.fori_loop(0, PH_CHUNKS // 2, pair, 0)
            for b in range(2):
                pltpu.make_async_copy(
                    srcs[b].at[src_v.at[0]], gbufs[b], sems_g[b]).wait()
                pltpu.make_async_copy(
                    sbufs[b], acc.at[dst_v.at[0]], sems_s[b]).wait()

        plsc.subcore_barrier()
        for t in range(ROWS_PER_TILE // ZROWS):
            lo = base + t * ZROWS
            pltpu.sync_copy(acc.at[pl.ds(lo, ZROWS)],
                            out_hbm.at[c, pl.ds(lo, ZROWS)])

    return k(T, src, dst, ew)


def _pack_table(X):
    return jnp.pad(X.astype(jnp.bfloat16), ((0, N_ACC - N_NODES), (0, 0)))


def _tc_body(p0_ref, p1_ref, w_ref, b_ref, o_ref):
    h = p0_ref[...].astype(jnp.float32) + p1_ref[...].astype(jnp.float32)
    o_ref[...] = (
        lax.dot_general(h, w_ref[...], (((1,), (1,)), ((), ())),
                        preferred_element_type=jnp.float32)
        + b_ref[...])


def _tc_linear(p0, p1, W, b2d):
    rows = 1000
    return pl.pallas_call(
        _tc_body,
        grid=(N_NODES // rows,),
        in_specs=[
            pl.BlockSpec((rows, D), lambda i: (i, 0)),
            pl.BlockSpec((rows, D), lambda i: (i, 0)),
            pl.BlockSpec((D, D), lambda i: (0, 0)),
            pl.BlockSpec((1, D), lambda i: (0, 0)),
        ],
        out_specs=pl.BlockSpec((rows, D), lambda i: (i, 0)),
        out_shape=jax.ShapeDtypeStruct((N_NODES, D), jnp.float32),
    )(p0, p1, W, b2d)


def kernel(X, edge_index, edge_weight, W, b):
    src = edge_index[1].astype(jnp.int32)
    dst = edge_index[0].astype(jnp.int32)
    wu16 = jax.lax.bitcast_convert_type(
        edge_weight.astype(jnp.bfloat16), jnp.uint16).astype(jnp.uint32)
    ew = wu16 | (wu16 << 16)
    pad = E_PAD - src.shape[0]
    src = jnp.pad(src, (0, pad)).reshape(NW, N_CHUNKS, CHUNK)
    dst = jnp.pad(dst, (0, pad)).reshape(NW, N_CHUNKS, CHUNK)
    ew = jnp.pad(ew, (0, pad)).reshape(NW, N_CHUNKS, CHUNK)
    part = _sc_scatter(_pack_table(X), src, dst, ew)
    return _tc_linear(part[0, :N_NODES], part[1, :N_NODES], W,
                      b.reshape(1, D))


# speedup vs baseline: 1.0912x; 1.0912x over previous
import functools

import jax
import jax.numpy as jnp
from jax import lax
from jax.experimental import pallas as pl
from jax.experimental.pallas import tpu as pltpu
from jax.experimental.pallas import tpu_sc as plsc

N_NODES = 10000
D = 128
DP = D // 2
NC = 2
NS = 16
NW = NC * NS
CHUNK = 128
N_CHUNKS = 80
N_PHASES = 2
PH_CHUNKS = N_CHUNKS // N_PHASES
E_PAD = NW * N_CHUNKS * CHUNK
N_ACC = 10240
ROWS_PER_TILE = N_ACC // NS
ZROWS = 128


def _sc_scatter(T, src, dst, ew):
    mesh = plsc.VectorSubcoreMesh(
        core_axis_name="c", subcore_axis_name="s",
        num_cores=NC, num_subcores=NS)

    @functools.partial(
        pl.kernel,
        out_type=jax.ShapeDtypeStruct((NC, N_ACC, D), jnp.bfloat16),
        mesh=mesh,
        scratch_types=[
            pltpu.VMEM((PH_CHUNKS, CHUNK), jnp.int32),
            pltpu.VMEM((PH_CHUNKS, CHUNK), jnp.int32),
            pltpu.VMEM((PH_CHUNKS, CHUNK), jnp.uint32),
            pltpu.VMEM((CHUNK, D), jnp.bfloat16),
            pltpu.VMEM((CHUNK, D), jnp.bfloat16),
            pltpu.VMEM((CHUNK, D), jnp.bfloat16),
            pltpu.VMEM((CHUNK, D), jnp.bfloat16),
            pltpu.VMEM_SHARED((N_ACC, D), jnp.bfloat16),
            pltpu.VMEM_SHARED((N_ACC, D), jnp.bfloat16),
            pltpu.SemaphoreType.DMA,
            pltpu.SemaphoreType.DMA,
            pltpu.SemaphoreType.DMA,
            pltpu.SemaphoreType.DMA,
        ],
        compiler_params=pltpu.CompilerParams(use_tc_tiling_on_sc=False,
                                             needs_layout_passes=False),
    )
    def k(t_hbm, src_hbm, dst_hbm, ew_hbm, out_hbm,
          src_v, dst_v, ew_v, g0, g1, s0, s1, tbl, acc,
          sem_g0, sem_g1, sem_s0, sem_s1):
        gbufs = (g0, g1)
        sbufs = (s0, s1)
        sems_g = (sem_g0, sem_g1)
        sems_s = (sem_s0, sem_s1)
        c = lax.axis_index("c")
        s = lax.axis_index("s")
        gwid = c * NS + s
        base = s * ROWS_PER_TILE

        pltpu.sync_copy(t_hbm.at[pl.ds(base, ROWS_PER_TILE)],
                        tbl.at[pl.ds(base, ROWS_PER_TILE)])

        def zrow(i, carry):
            for v in range(D // 32):
                s0[i, pl.ds(32 * v, 32)] = jnp.zeros((32,), jnp.bfloat16)
            return carry
        lax.fori_loop(0, ZROWS, zrow, 0)
        for t in range(ROWS_PER_TILE // ZROWS):
            pltpu.sync_copy(s0, acc.at[pl.ds(base + t * ZROWS, ZROWS)])
        plsc.subcore_barrier()

        def scale(j, src_buf, dst_buf):
            def group(g, gcarry):
                wv = ew_v[j, pl.ds(g * 16, 16)]
                ws = [plsc.bitcast(jnp.full((16,), wv[i], jnp.uint32),
                                   jnp.bfloat16)
                      for i in range(16)]

                def blk(v, bcarry):
                    psl = pl.ds(v * 32, 32)
                    for i in range(16):
                        e = g * 16 + i
                        dst_buf[e, psl] = src_buf[e, psl] * ws[i]
                    return bcarry
                lax.fori_loop(0, D // 32, blk, 0)
                return gcarry
            lax.fori_loop(0, CHUNK // 16, group, 0)

        for phase in range(N_PHASES):
            p0 = phase * PH_CHUNKS
            pltpu.sync_copy(src_hbm.at[gwid, pl.ds(p0, PH_CHUNKS)], src_v)
            pltpu.sync_copy(dst_hbm.at[gwid, pl.ds(p0, PH_CHUNKS)], dst_v)
            pltpu.sync_copy(ew_hbm.at[gwid, pl.ds(p0, PH_CHUNKS)], ew_v)

            for b in range(2):
                pltpu.async_copy(tbl.at[src_v.at[b]], gbufs[b], sems_g[b])

            def pair(q, carry):
                for b in range(2):
                    j = 2 * q + b
                    jn = jnp.minimum(j + 2, PH_CHUNKS - 1)

                    pltpu.make_async_copy(
                        tbl.at[src_v.at[j]], gbufs[b], sems_g[b]).wait()

                    @pl.when(j >= 2)
                    def _():
                        pltpu.make_async_copy(
                            sbufs[b], acc.at[dst_v.at[j]], sems_s[b]).wait()
                    scale(j, gbufs[b], sbufs[b])
                    pltpu.async_copy(sbufs[b], acc.at[dst_v.at[j]],
                                     sems_s[b], add=True)
                    pltpu.async_copy(tbl.at[src_v.at[jn]], gbufs[b],
                                     sems_g[b])
                return carry
            lax.fori_loop(0, PH_CHUNKS // 2, pair, 0)
            for b in range(2):
                pltpu.make_async_copy(
                    tbl.at[src_v.at[0]], gbufs[b], sems_g[b]).wait()
                pltpu.make_async_copy(
                    sbufs[b], acc.at[dst_v.at[0]], sems_s[b]).wait()

        plsc.subcore_barrier()
        for t in range(ROWS_PER_TILE // ZROWS):
            lo = base + t * ZROWS
            pltpu.sync_copy(acc.at[pl.ds(lo, ZROWS)],
                            out_hbm.at[c, pl.ds(lo, ZROWS)])

    return k(T, src, dst, ew)


def _pack_table(X):
    return jnp.pad(X.astype(jnp.bfloat16), ((0, N_ACC - N_NODES), (0, 0)))


def _tc_body(p0_ref, p1_ref, w_ref, b_ref, o_ref):
    h = p0_ref[...].astype(jnp.float32) + p1_ref[...].astype(jnp.float32)
    o_ref[...] = (
        lax.dot_general(h, w_ref[...], (((1,), (1,)), ((), ())),
                        preferred_element_type=jnp.float32)
        + b_ref[...])


def _tc_linear(p0, p1, W, b2d):
    rows = 1000
    return pl.pallas_call(
        _tc_body,
        grid=(N_NODES // rows,),
        in_specs=[
            pl.BlockSpec((rows, D), lambda i: (i, 0)),
            pl.BlockSpec((rows, D), lambda i: (i, 0)),
            pl.BlockSpec((D, D), lambda i: (0, 0)),
            pl.BlockSpec((1, D), lambda i: (0, 0)),
        ],
        out_specs=pl.BlockSpec((rows, D), lambda i: (i, 0)),
        out_shape=jax.ShapeDtypeStruct((N_NODES, D), jnp.float32),
    )(p0, p1, W, b2d)


def kernel(X, edge_index, edge_weight, W, b):
    src = edge_index[1].astype(jnp.int32)
    dst = edge_index[0].astype(jnp.int32)
    wu16 = jax.lax.bitcast_convert_type(
        edge_weight.astype(jnp.bfloat16), jnp.uint16).astype(jnp.uint32)
    ew = wu16 | (wu16 << 16)
    pad = E_PAD - src.shape[0]
    src = jnp.pad(src, (0, pad)).reshape(NW, N_CHUNKS, CHUNK)
    dst = jnp.pad(dst, (0, pad)).reshape(NW, N_CHUNKS, CHUNK)
    ew = jnp.pad(ew, (0, pad)).reshape(NW, N_CHUNKS, CHUNK)
    part = _sc_scatter(_pack_table(X), src, dst, ew)
    return _tc_linear(part[0, :N_NODES], part[1, :N_NODES], W,
                      b.reshape(1, D))
